# Initial kernel scaffold; baseline (speedup 1.0000x reference)
import jax
import jax.numpy as jnp
from jax.experimental import pallas as pl


def kernel(idx, embedding, nan_mask, impute_values, missing_vector):
    raise NotImplementedError("write your pallas kernel here")


# speedup vs baseline: 1.6344x; 1.6344x over previous
import functools

import jax
import jax.numpy as jnp
from jax import lax
from jax.experimental import pallas as pl
from jax.experimental.pallas import tpu as pltpu
from jax.experimental.pallas import tpu_sc as plsc

_VOCAB = 1000000
_DIM = 64
_L = 16

_INFO = plsc.get_sparse_core_info()
_NC = _INFO.num_cores
_NS = _INFO.num_subcores
_NW = _NC * _NS

_B_TOTAL = 16384 * 50
_B_PER_W = _B_TOTAL // _NW
_CHUNK = 512
_GRP = 128
_NG = _CHUNK // _GRP
_ITERS = _B_PER_W // _CHUNK


def _sc_body(emb_hbm, msk_hbm, idx_hbm, imp_hbm, mv_hbm, out_hbm,
             idx_v, g_v, emb_v, msk_v, imp_v, mv_v, sem_e, sem_m):
    wid = lax.axis_index("s") * _NC + lax.axis_index("c")
    base = wid * _B_PER_W

    pltpu.sync_copy(imp_hbm, imp_v)
    pltpu.sync_copy(mv_hbm, mv_v)

    def chunk_body(t, carry):
        row0 = base + t * _CHUNK
        pltpu.sync_copy(idx_hbm.at[pl.ds(row0, _CHUNK)], idx_v)

        for i in range(_CHUNK // _L):
            v = idx_v[pl.ds(i * _L, _L)]
            g = jnp.where(v == 0, _VOCAB - 1, v - 1)
            r, c = i // (_GRP // _L), i % (_GRP // _L)
            g_v[r, pl.ds(c * _L, _L)] = g

        handles = []
        for r in range(_NG):
            handles.append(pltpu.async_copy(
                emb_hbm.at[g_v.at[r]], emb_v.at[pl.ds(r * _GRP, _GRP)],
                sem_e))
            handles.append(pltpu.async_copy(
                msk_hbm.at[g_v.at[r]], msk_v.at[pl.ds(r * _GRP, _GRP)],
                sem_m))
        for h in handles:
            h.wait()

        park = tuple(imp_v[pl.ds(16 * j, 16)] for j in range(4)) + \
            tuple(mv_v[pl.ds(16 * j, 16)] for j in range(4))

        def row_body(b, carry2):
            impc = carry2[0:4]
            mvc = carry2[4:8]
            mrow = msk_v[b, pl.ds(0, 16)]
            for j in range(4):
                e = emb_v[b, pl.ds(16 * j, 16)]
                bj = lax.shift_right_logical(mrow, 8 * j) & 3
                impf = (bj & 1).astype(jnp.float32)
                missf = lax.shift_right_logical(bj, 1).astype(jnp.float32)
                e = e + (impc[j] - e) * impf
                e = e + (mvc[j] - e) * missf
                emb_v[b, pl.ds(16 * j, 16)] = e
            return carry2

        lax.fori_loop(0, _CHUNK, row_body, park, unroll=False)

        pltpu.sync_copy(emb_v, out_hbm.at[pl.ds(row0, _CHUNK)])
        return carry

    lax.fori_loop(0, _ITERS, chunk_body, 0, unroll=False)


@jax.jit
def _sc_gather(emb, msk_w, idx_flat, imp, mv):
    mesh = plsc.VectorSubcoreMesh(core_axis_name="c", subcore_axis_name="s")
    fn = pl.kernel(
        _sc_body,
        mesh=mesh,
        compiler_params=pltpu.CompilerParams(use_tc_tiling_on_sc=False),
        out_type=jax.ShapeDtypeStruct((_B_TOTAL, _DIM), jnp.float32),
        scratch_types=[
            pltpu.VMEM((_CHUNK,), jnp.int32),
            pltpu.VMEM((_NG, _GRP), jnp.int32),
            pltpu.VMEM((_CHUNK, _DIM), jnp.float32),
            pltpu.VMEM((_CHUNK, 16), jnp.int32),
            pltpu.VMEM((_DIM,), jnp.float32),
            pltpu.VMEM((_DIM,), jnp.float32),
            pltpu.SemaphoreType.DMA,
            pltpu.SemaphoreType.DMA,
        ],
    )
    return fn(emb, msk_w, idx_flat, imp, mv)


def kernel(idx, embedding, nan_mask, impute_values, missing_vector):
    idx_flat = idx.reshape(-1).astype(jnp.int32)
    mask_u8 = nan_mask.astype(jnp.uint8)
    mask_w = lax.bitcast_convert_type(
        mask_u8.reshape(_VOCAB, 4, 16).transpose(0, 2, 1), jnp.int32)
    mask_w = mask_w.at[_VOCAB - 1].set(
        jnp.full((16,), 0x02020202, jnp.int32))
    imp = impute_values.astype(jnp.float32)
    mv = missing_vector.reshape(-1).astype(jnp.float32)
    out = _sc_gather(embedding, mask_w, idx_flat, imp, mv)
    return out.reshape(idx.shape + (_DIM,))
